# baseline (speedup 1.0000x reference)
import jax
import jax.numpy as jnp
from jax.experimental import pallas as pl
from jax.experimental.pallas import tpu as pltpu

_CH = 8


def _gcn_body(x_hbm, adj_ref, w_ref, b_ref, out_hbm, xbuf, obuf, lsem, ssem):
    adj = adj_ref[...]
    C = adj.shape[0]
    deg_r = jnp.sum(adj, axis=1, keepdims=True) + 1.0
    deg_c = jnp.sum(adj, axis=0, keepdims=True) + 1.0
    dinv_r = jax.lax.rsqrt(deg_r)
    dinv_c = jax.lax.rsqrt(deg_c)
    eye = jnp.eye(C, dtype=adj.dtype)
    A = (adj + eye) * dinv_r * dinv_c * w_ref[0, 0]
    bias = b_ref[0, 0]

    nck = x_hbm.shape[0] // _CH

    def load(i):
        return pltpu.make_async_copy(
            x_hbm.at[pl.ds(i * _CH, _CH)], xbuf.at[i % 2], lsem.at[i % 2])

    def store(i):
        return pltpu.make_async_copy(
            obuf.at[i % 2], out_hbm.at[pl.ds(i * _CH, _CH)], ssem.at[i % 2])

    load(0).start()
    for i in range(nck):
        load(i).wait()
        if i + 1 < nck:
            load(i + 1).start()
        if i >= 2:
            store(i - 2).wait()
        slot = i % 2
        for j in range(_CH):
            obuf[slot, j] = jax.lax.dot_general(
                A, xbuf[slot, j], (((1,), (0,)), ((), ())),
                preferred_element_type=jnp.float32) + bias
        store(i).start()
    store(nck - 2).wait()
    store(nck - 1).wait()


def kernel(x, adj, W, b):
    B, C, T = x.shape
    out = pl.pallas_call(
        _gcn_body,
        in_specs=[
            pl.BlockSpec(memory_space=pl.ANY),
            pl.BlockSpec(memory_space=pltpu.VMEM),
            pl.BlockSpec(memory_space=pltpu.VMEM),
            pl.BlockSpec(memory_space=pltpu.VMEM),
        ],
        out_specs=pl.BlockSpec(memory_space=pl.ANY),
        out_shape=jax.ShapeDtypeStruct((B, C, T), jnp.float32),
        scratch_shapes=[
            pltpu.VMEM((2, _CH, C, T), jnp.float32),
            pltpu.VMEM((2, _CH, C, T), jnp.float32),
            pltpu.SemaphoreType.DMA((2,)),
            pltpu.SemaphoreType.DMA((2,)),
        ],
    )(x, adj, W, b.reshape(1, 1))
    return out


# speedup vs baseline: 1.7103x; 1.7103x over previous
import jax
import jax.numpy as jnp
from jax.experimental import pallas as pl
from jax.experimental.pallas import tpu as pltpu

_CH = 8


def _gcn_body(x_hbm, adj_ref, w_ref, b_ref, out_hbm, xbuf, obuf, lsem, ssem):
    adj = adj_ref[...]
    C = adj.shape[0]
    deg_r = jnp.sum(adj, axis=1, keepdims=True) + 1.0
    deg_c = jnp.sum(adj, axis=0, keepdims=True) + 1.0
    dinv_r = jax.lax.rsqrt(deg_r)
    dinv_c = jax.lax.rsqrt(deg_c)
    eye = jnp.eye(C, dtype=adj.dtype)
    A = (adj + eye) * dinv_r * dinv_c * w_ref[0, 0]
    bias = b_ref[0, 0]

    nck = x_hbm.shape[0] // _CH

    def load(i):
        return pltpu.make_async_copy(
            x_hbm.at[pl.ds(i * _CH, _CH)], xbuf.at[i], lsem.at[i])

    def store(i):
        return pltpu.make_async_copy(
            obuf.at[i], out_hbm.at[pl.ds(i * _CH, _CH)], ssem.at[i])

    for i in range(nck):
        load(i).start()
    for i in range(nck):
        load(i).wait()
        for j in range(_CH):
            obuf[i, j] = jax.lax.dot_general(
                A, xbuf[i, j], (((1,), (0,)), ((), ())),
                preferred_element_type=jnp.float32) + bias
        store(i).start()
    for i in range(nck):
        store(i).wait()


def kernel(x, adj, W, b):
    B, C, T = x.shape
    out = pl.pallas_call(
        _gcn_body,
        in_specs=[
            pl.BlockSpec(memory_space=pl.ANY),
            pl.BlockSpec(memory_space=pltpu.VMEM),
            pl.BlockSpec(memory_space=pltpu.VMEM),
            pl.BlockSpec(memory_space=pltpu.VMEM),
        ],
        out_specs=pl.BlockSpec(memory_space=pl.ANY),
        out_shape=jax.ShapeDtypeStruct((B, C, T), jnp.float32),
        scratch_shapes=[
            pltpu.VMEM((B // _CH, _CH, C, T), jnp.float32),
            pltpu.VMEM((B // _CH, _CH, C, T), jnp.float32),
            pltpu.SemaphoreType.DMA((B // _CH,)),
            pltpu.SemaphoreType.DMA((B // _CH,)),
        ],
    )(x, adj, W, b.reshape(1, 1))
    return out
